# baseline (speedup 1.0000x reference)
import functools

import jax
import jax.numpy as jnp
from jax import lax
from jax.experimental import pallas as pl
from jax.experimental.pallas import tpu as pltpu
from jax.experimental.pallas import tpu_sc as plsc

BATCH = 16384
DIM = 16
ODIM = DIM + 1
NUM_CORES = 2
NUM_SUBCORES = 16
NW = NUM_CORES * NUM_SUBCORES
BPW = BATCH // NW


def _sc_body(users, items, uid, iid, out_u, out_i,
             uidx, iidx, ug, ig, ubuf, ibuf, sem_u, sem_i):
    wid = lax.axis_index("s") * NUM_CORES + lax.axis_index("c")
    base = wid * BPW

    pltpu.sync_copy(uid.at[pl.ds(base, BPW)], uidx)
    pltpu.sync_copy(iid.at[pl.ds(base, BPW)], iidx)

    cu = pltpu.async_copy(users.at[uidx], ug, sem_u)
    ci = pltpu.async_copy(items.at[iidx], ig, sem_i)

    iota = lax.iota(jnp.int32, 16)
    for j in range(BPW // 16):
        sl = pl.ds(j * 16, 16)
        pos = (j * 16 + iota) * ODIM
        plsc.store_scatter(ubuf, [pos], uidx[sl].astype(jnp.float32))
        plsc.store_scatter(ibuf, [pos], iidx[sl].astype(jnp.float32))

    cu.wait()
    ci.wait()

    def row(r, _):
        dst = r * ODIM + 1 + iota
        plsc.store_scatter(ubuf, [dst], ug[r, :])
        plsc.store_scatter(ibuf, [dst], ig[r, :])
        return 0

    lax.fori_loop(0, BPW, row, 0, unroll=8)

    pltpu.sync_copy(ubuf, out_u.at[pl.ds(base * ODIM, BPW * ODIM)])
    pltpu.sync_copy(ibuf, out_i.at[pl.ds(base * ODIM, BPW * ODIM)])


@functools.partial(
    pl.kernel,
    out_type=(
        jax.ShapeDtypeStruct((BATCH * ODIM,), jnp.float32),
        jax.ShapeDtypeStruct((BATCH * ODIM,), jnp.float32),
    ),
    mesh=plsc.VectorSubcoreMesh(core_axis_name="c", subcore_axis_name="s"),
    compiler_params=pltpu.CompilerParams(use_tc_tiling_on_sc=False,
                                         needs_layout_passes=False),
    scratch_types=[
        pltpu.VMEM((BPW,), jnp.int32),
        pltpu.VMEM((BPW,), jnp.int32),
        pltpu.VMEM((BPW, DIM), jnp.float32),
        pltpu.VMEM((BPW, DIM), jnp.float32),
        pltpu.VMEM((BPW * ODIM,), jnp.float32),
        pltpu.VMEM((BPW * ODIM,), jnp.float32),
        pltpu.SemaphoreType.DMA,
        pltpu.SemaphoreType.DMA,
    ],
)
def _sc_lookup(users, items, uid, iid, out_u, out_i,
               uidx, iidx, ug, ig, ubuf, ibuf, sem_u, sem_i):
    _sc_body(users, items, uid, iid, out_u, out_i,
             uidx, iidx, ug, ig, ubuf, ibuf, sem_u, sem_i)


def kernel(users, items, user_id, item_id):
    uid = user_id.astype(jnp.int32)
    iid = item_id.astype(jnp.int32)
    fu, fi = _sc_lookup(users, items, uid, iid)
    return (fu.reshape(BATCH, ODIM), fi.reshape(BATCH, ODIM))


# speedup vs baseline: 5.3202x; 5.3202x over previous
import functools

import jax
import jax.numpy as jnp
from jax import lax
from jax.experimental import pallas as pl
from jax.experimental.pallas import tpu as pltpu
from jax.experimental.pallas import tpu_sc as plsc

BATCH = 16384
DIM = 16
ODIM = DIM + 1
NUM_CORES = 2
NUM_SUBCORES = 16
NW = NUM_CORES * NUM_SUBCORES
BPW = BATCH // NW
K = 16


def _do_table(tab, idv, obuf, slabs, sem):
    iota = lax.iota(jnp.int32, 16)

    @pl.loop(0, BPW // 16)
    def _fill_ids(j):
        sl = pl.ds(pl.multiple_of(j * 16, 16), 16)
        obuf[0, sl] = idv[sl].astype(jnp.float32)

    @pl.loop(0, BPW, step=K)
    def _group(g):
        sl = pl.ds(pl.multiple_of(g, K), K)
        rv = idv[sl]
        offv = lax.shift_right_logical(rv, jnp.full((16,), 7, jnp.int32))
        offv = offv * 128
        lanev = lax.bitwise_and(rv, jnp.full((16,), 127, jnp.int32))
        copies = []
        for k in range(K):
            off = lax.reduce_max(jnp.where(iota == k, offv, 0), (0,))
            off = pl.multiple_of(off, 128)
            copies.append(pltpu.async_copy(
                tab.at[:, pl.ds(off, 128)], slabs.at[k], sem))
        for k in range(K):
            copies[k].wait()
            lane_k = lax.reduce_max(jnp.where(iota == k, lanev, 0), (0,))
            lane = jnp.full((16,), lane_k, jnp.int32)
            v = plsc.load_gather(slabs.at[k], [iota, lane])
            col = g + k + jnp.zeros((16,), jnp.int32)
            plsc.store_scatter(obuf, [1 + iota, col], v)


def _sc_body(ut, it, uid, iid, out_u, out_i,
             uidv, iidv, slabs, obu, obi, sem):
    wid = lax.axis_index("s") * NUM_CORES + lax.axis_index("c")
    base = wid * BPW

    pltpu.sync_copy(uid.at[pl.ds(base, BPW)], uidv)
    pltpu.sync_copy(iid.at[pl.ds(base, BPW)], iidv)

    _do_table(ut, uidv, obu, slabs, sem)
    _do_table(it, iidv, obi, slabs, sem)

    pltpu.sync_copy(obu, out_u.at[:, pl.ds(base, BPW)])
    pltpu.sync_copy(obi, out_i.at[:, pl.ds(base, BPW)])


@functools.partial(
    pl.kernel,
    out_type=(
        jax.ShapeDtypeStruct((ODIM, BATCH), jnp.float32),
        jax.ShapeDtypeStruct((ODIM, BATCH), jnp.float32),
    ),
    mesh=plsc.VectorSubcoreMesh(core_axis_name="c", subcore_axis_name="s"),
    compiler_params=pltpu.CompilerParams(needs_layout_passes=False),
    scratch_types=[
        pltpu.VMEM((BPW,), jnp.int32),
        pltpu.VMEM((BPW,), jnp.int32),
        pltpu.VMEM((K, DIM, 128), jnp.float32),
        pltpu.VMEM((ODIM, BPW), jnp.float32),
        pltpu.VMEM((ODIM, BPW), jnp.float32),
        pltpu.SemaphoreType.DMA,
    ],
)
def _sc_lookup(ut, it, uid, iid, out_u, out_i,
               uidv, iidv, slabs, obu, obi, sem):
    _sc_body(ut, it, uid, iid, out_u, out_i,
             uidv, iidv, slabs, obu, obi, sem)


def kernel(users, items, user_id, item_id):
    uid = user_id.astype(jnp.int32)
    iid = item_id.astype(jnp.int32)
    ou, oi = _sc_lookup(users.T, items.T, uid, iid)
    return (ou.T, oi.T)


# speedup vs baseline: 5.7680x; 1.0842x over previous
import functools

import jax
import jax.numpy as jnp
from jax import lax
from jax.experimental import pallas as pl
from jax.experimental.pallas import tpu as pltpu
from jax.experimental.pallas import tpu_sc as plsc

BATCH = 16384
DIM = 16
ODIM = DIM + 1
NUM_CORES = 2
NUM_SUBCORES = 16
NW = NUM_CORES * NUM_SUBCORES
BPW = BATCH // NW
K = 16


NG = BPW // K


def _fire(tab, idv, slabs, b, sem, g):
    iota = lax.iota(jnp.int32, 16)
    sl = pl.ds(pl.multiple_of(g * K, K), K)
    rv = idv[sl]
    offv = lax.shift_right_logical(rv, jnp.full((16,), 7, jnp.int32)) * 128
    copies = []
    for k in range(K):
        off = lax.reduce_max(jnp.where(iota == k, offv, 0), (0,))
        off = pl.multiple_of(off, 128)
        copies.append(pltpu.async_copy(
            tab.at[:, pl.ds(off, 128)], slabs.at[b, k], sem))
    return copies


def _wait_group(tab, slabs, b, sem):
    for k in range(K):
        pltpu.make_async_copy(
            tab.at[:, pl.ds(0, 128)], slabs.at[b, k], sem).wait()


def _extract(idv, obuf, slabs, b, g):
    iota = lax.iota(jnp.int32, 16)
    sl = pl.ds(pl.multiple_of(g * K, K), K)
    lanev = lax.bitwise_and(idv[sl], jnp.full((16,), 127, jnp.int32))
    for k in range(K):
        lane_k = lax.reduce_max(jnp.where(iota == k, lanev, 0), (0,))
        lane = jnp.full((16,), lane_k, jnp.int32)
        v = plsc.load_gather(slabs.at[b, k], [iota, lane])
        col = g * K + k + jnp.zeros((16,), jnp.int32)
        plsc.store_scatter(obuf, [1 + iota, col], v)


def _do_table(tab, idv, obuf, slabs, sem_a, sem_b):
    @pl.loop(0, BPW // 16)
    def _fill_ids(j):
        sl = pl.ds(pl.multiple_of(j * 16, 16), 16)
        obuf[0, sl] = idv[sl].astype(jnp.float32)

    _fire(tab, idv, slabs, 0, sem_a, 0)

    @pl.loop(0, NG, step=2)
    def _pair(g):
        _fire(tab, idv, slabs, 1, sem_b, jnp.minimum(g + 1, NG - 1))
        _wait_group(tab, slabs, 0, sem_a)
        _extract(idv, obuf, slabs, 0, g)
        _fire(tab, idv, slabs, 0, sem_a, jnp.minimum(g + 2, NG - 1))
        _wait_group(tab, slabs, 1, sem_b)
        _extract(idv, obuf, slabs, 1, g + 1)

    _wait_group(tab, slabs, 0, sem_a)


def _sc_body(ut, it, uid, iid, out_u, out_i,
             uidv, iidv, slabs, obu, obi, sem_a, sem_b):
    wid = lax.axis_index("s") * NUM_CORES + lax.axis_index("c")
    base = wid * BPW

    pltpu.sync_copy(uid.at[pl.ds(base, BPW)], uidv)
    pltpu.sync_copy(iid.at[pl.ds(base, BPW)], iidv)

    _do_table(ut, uidv, obu, slabs, sem_a, sem_b)
    _do_table(it, iidv, obi, slabs, sem_a, sem_b)

    pltpu.sync_copy(obu, out_u.at[:, pl.ds(base, BPW)])
    pltpu.sync_copy(obi, out_i.at[:, pl.ds(base, BPW)])


@functools.partial(
    pl.kernel,
    out_type=(
        jax.ShapeDtypeStruct((ODIM, BATCH), jnp.float32),
        jax.ShapeDtypeStruct((ODIM, BATCH), jnp.float32),
    ),
    mesh=plsc.VectorSubcoreMesh(core_axis_name="c", subcore_axis_name="s"),
    compiler_params=pltpu.CompilerParams(needs_layout_passes=False),
    scratch_types=[
        pltpu.VMEM((BPW,), jnp.int32),
        pltpu.VMEM((BPW,), jnp.int32),
        pltpu.VMEM((2, K, DIM, 128), jnp.float32),
        pltpu.VMEM((ODIM, BPW), jnp.float32),
        pltpu.VMEM((ODIM, BPW), jnp.float32),
        pltpu.SemaphoreType.DMA,
        pltpu.SemaphoreType.DMA,
    ],
)
def _sc_lookup(ut, it, uid, iid, out_u, out_i,
               uidv, iidv, slabs, obu, obi, sem_a, sem_b):
    _sc_body(ut, it, uid, iid, out_u, out_i,
             uidv, iidv, slabs, obu, obi, sem_a, sem_b)


def kernel(users, items, user_id, item_id):
    uid = user_id.astype(jnp.int32)
    iid = item_id.astype(jnp.int32)
    ou, oi = _sc_lookup(users.T, items.T, uid, iid)
    return (ou.T, oi.T)


# speedup vs baseline: 5.9654x; 1.0342x over previous
import functools

import jax
import jax.numpy as jnp
from jax import lax
from jax.experimental import pallas as pl
from jax.experimental.pallas import tpu as pltpu
from jax.experimental.pallas import tpu_sc as plsc

BATCH = 16384
DIM = 16
ODIM = DIM + 1
NUM_CORES = 2
NUM_SUBCORES = 16
NW = NUM_CORES * NUM_SUBCORES
BPW = BATCH // NW
K = 16


NG = BPW // K


def _fire(tab, idv, slabs, b, sem, g):
    iota = lax.iota(jnp.int32, 16)
    sl = pl.ds(pl.multiple_of(g * K, K), K)
    rv = idv[sl]
    offv = lax.shift_right_logical(rv, jnp.full((16,), 7, jnp.int32)) * 128
    for k in range(K):
        off = lax.reduce_max(jnp.where(iota == k, offv, 0), (0,))
        off = pl.multiple_of(off, 128)
        pltpu.async_copy(tab.at[:, pl.ds(off, 128)], slabs.at[b, k], sem)


def _wait_group(tab, slabs, b, sem):
    for k in range(K):
        pltpu.make_async_copy(
            tab.at[:, pl.ds(0, 128)], slabs.at[b, k], sem).wait()


def _extract(idv, obuf, slabs, b, g):
    iota = lax.iota(jnp.int32, 16)
    sl = pl.ds(pl.multiple_of(g * K, K), K)
    lanev = lax.bitwise_and(idv[sl], jnp.full((16,), 127, jnp.int32))
    for f in range(DIM):
        fv = jnp.full((16,), f, jnp.int32)
        v = plsc.load_gather(slabs.at[b], [iota, fv, lanev])
        obuf[1 + f, sl] = v


def _do_table(tab, idv, obuf, slabs, sem_a, sem_b):
    @pl.loop(0, BPW // 16)
    def _fill_ids(j):
        sl = pl.ds(pl.multiple_of(j * 16, 16), 16)
        obuf[0, sl] = idv[sl].astype(jnp.float32)

    _fire(tab, idv, slabs, 0, sem_a, 0)

    @pl.loop(0, NG, step=2)
    def _pair(g):
        _fire(tab, idv, slabs, 1, sem_b, jnp.minimum(g + 1, NG - 1))
        _wait_group(tab, slabs, 0, sem_a)
        _extract(idv, obuf, slabs, 0, g)
        _fire(tab, idv, slabs, 0, sem_a, jnp.minimum(g + 2, NG - 1))
        _wait_group(tab, slabs, 1, sem_b)
        _extract(idv, obuf, slabs, 1, g + 1)

    _wait_group(tab, slabs, 0, sem_a)


def _sc_body(ut, it, uid, iid, out_u, out_i,
             uidv, iidv, slabs, obu, obi, sem_a, sem_b):
    wid = lax.axis_index("s") * NUM_CORES + lax.axis_index("c")
    base = wid * BPW

    pltpu.sync_copy(uid.at[pl.ds(base, BPW)], uidv)
    pltpu.sync_copy(iid.at[pl.ds(base, BPW)], iidv)

    _do_table(ut, uidv, obu, slabs, sem_a, sem_b)
    _do_table(it, iidv, obi, slabs, sem_a, sem_b)

    pltpu.sync_copy(obu, out_u.at[:, pl.ds(base, BPW)])
    pltpu.sync_copy(obi, out_i.at[:, pl.ds(base, BPW)])


@functools.partial(
    pl.kernel,
    out_type=(
        jax.ShapeDtypeStruct((ODIM, BATCH), jnp.float32),
        jax.ShapeDtypeStruct((ODIM, BATCH), jnp.float32),
    ),
    mesh=plsc.VectorSubcoreMesh(core_axis_name="c", subcore_axis_name="s"),
    compiler_params=pltpu.CompilerParams(needs_layout_passes=False),
    scratch_types=[
        pltpu.VMEM((BPW,), jnp.int32),
        pltpu.VMEM((BPW,), jnp.int32),
        pltpu.VMEM((2, K, DIM, 128), jnp.float32),
        pltpu.VMEM((ODIM, BPW), jnp.float32),
        pltpu.VMEM((ODIM, BPW), jnp.float32),
        pltpu.SemaphoreType.DMA,
        pltpu.SemaphoreType.DMA,
    ],
)
def _sc_lookup(ut, it, uid, iid, out_u, out_i,
               uidv, iidv, slabs, obu, obi, sem_a, sem_b):
    _sc_body(ut, it, uid, iid, out_u, out_i,
             uidv, iidv, slabs, obu, obi, sem_a, sem_b)


def kernel(users, items, user_id, item_id):
    uid = user_id.astype(jnp.int32)
    iid = item_id.astype(jnp.int32)
    ou, oi = _sc_lookup(users.T, items.T, uid, iid)
    return (ou.T, oi.T)


# speedup vs baseline: 6.2245x; 1.0434x over previous
import functools

import jax
import jax.numpy as jnp
from jax import lax
from jax.experimental import pallas as pl
from jax.experimental.pallas import tpu as pltpu
from jax.experimental.pallas import tpu_sc as plsc

BATCH = 16384
DIM = 16
ODIM = DIM + 1
NUM_CORES = 2
NUM_SUBCORES = 16
NW = NUM_CORES * NUM_SUBCORES
BPW = BATCH // NW
K = 16


NG = BPW // K


def _fire(tab, idv, slabs, b, sem, g):
    iota = lax.iota(jnp.int32, 16)
    sl = pl.ds(pl.multiple_of(g * K, K), K)
    rv = idv[sl]
    offv = lax.shift_right_logical(rv, jnp.full((16,), 7, jnp.int32)) * 128
    for k in range(K):
        off = lax.reduce_max(jnp.where(iota == k, offv, 0), (0,))
        off = pl.multiple_of(off, 128)
        pltpu.async_copy(tab.at[:, pl.ds(off, 128)], slabs.at[b, k], sem)


def _wait_group(tab, slabs, b, sem):
    for k in range(K):
        pltpu.make_async_copy(
            tab.at[:, pl.ds(0, 128)], slabs.at[b, k], sem).wait()


def _extract(idv, obuf, slabs, b, g):
    iota = lax.iota(jnp.int32, 16)
    sl = pl.ds(pl.multiple_of(g * K, K), K)
    lanev = lax.bitwise_and(idv[sl], jnp.full((16,), 127, jnp.int32))
    for f in range(DIM):
        fv = jnp.full((16,), f, jnp.int32)
        v = plsc.load_gather(slabs.at[b], [iota, fv, lanev])
        obuf[1 + f, sl] = v


def _do_table(tab, idv, obuf, slabs, sems):
    @pl.loop(0, BPW // 16)
    def _fill_ids(j):
        sl = pl.ds(pl.multiple_of(j * 16, 16), 16)
        obuf[0, sl] = idv[sl].astype(jnp.float32)

    for b in range(3):
        _fire(tab, idv, slabs, b, sems[b], b)

    @pl.loop(0, NG - 2, step=3)
    def _triple(g):
        for j in range(3):
            _wait_group(tab, slabs, j, sems[j])
            _extract(idv, obuf, slabs, j, g + j)
            _fire(tab, idv, slabs, j, sems[j],
                  jnp.minimum(g + j + 3, NG - 1))

    _wait_group(tab, slabs, 0, sems[0])
    _extract(idv, obuf, slabs, 0, NG - 2)
    _wait_group(tab, slabs, 1, sems[1])
    _extract(idv, obuf, slabs, 1, NG - 1)
    _wait_group(tab, slabs, 2, sems[2])


def _sc_body(ut, it, uid, iid, out_u, out_i,
             uidv, iidv, slabs, obu, obi, sem_a, sem_b, sem_c):
    wid = lax.axis_index("s") * NUM_CORES + lax.axis_index("c")
    base = wid * BPW

    pltpu.sync_copy(uid.at[pl.ds(base, BPW)], uidv)
    pltpu.sync_copy(iid.at[pl.ds(base, BPW)], iidv)

    sems = (sem_a, sem_b, sem_c)
    _do_table(ut, uidv, obu, slabs, sems)
    _do_table(it, iidv, obi, slabs, sems)

    pltpu.sync_copy(obu, out_u.at[:, pl.ds(base, BPW)])
    pltpu.sync_copy(obi, out_i.at[:, pl.ds(base, BPW)])


@functools.partial(
    pl.kernel,
    out_type=(
        jax.ShapeDtypeStruct((ODIM, BATCH), jnp.float32),
        jax.ShapeDtypeStruct((ODIM, BATCH), jnp.float32),
    ),
    mesh=plsc.VectorSubcoreMesh(core_axis_name="c", subcore_axis_name="s"),
    compiler_params=pltpu.CompilerParams(needs_layout_passes=False),
    scratch_types=[
        pltpu.VMEM((BPW,), jnp.int32),
        pltpu.VMEM((BPW,), jnp.int32),
        pltpu.VMEM((3, K, DIM, 128), jnp.float32),
        pltpu.VMEM((ODIM, BPW), jnp.float32),
        pltpu.VMEM((ODIM, BPW), jnp.float32),
        pltpu.SemaphoreType.DMA,
        pltpu.SemaphoreType.DMA,
        pltpu.SemaphoreType.DMA,
    ],
)
def _sc_lookup(ut, it, uid, iid, out_u, out_i,
               uidv, iidv, slabs, obu, obi, sem_a, sem_b, sem_c):
    _sc_body(ut, it, uid, iid, out_u, out_i,
             uidv, iidv, slabs, obu, obi, sem_a, sem_b, sem_c)


def kernel(users, items, user_id, item_id):
    uid = user_id.astype(jnp.int32)
    iid = item_id.astype(jnp.int32)
    ou, oi = _sc_lookup(users.T, items.T, uid, iid)
    return (ou.T, oi.T)
